# baseline (speedup 1.0000x reference)
import functools

import jax
import jax.numpy as jnp
from jax import lax
from jax.experimental import pallas as pl
from jax.experimental.pallas import tpu as pltpu
from jax.experimental.pallas import tpu_sc as plsc

N = 10000
D = 128
E = 320000
NC = 2
NS = 16
NW = NC * NS
CH = 80
NCH = 128
BCH = 8
NBL = NCH // BCH
EPW = NCH * CH
EP = NW * EPW
NP = 10240
RPT = NP // NS
VG = D // 16
NB = 4
ZC = RPT // CH


def _spmm_body(x_hbm, pk_hbm, wt_hbm, out_hbm, pk_v, wt_v, rows_v, acc, *sems):
    si = sems[0:2]
    sg = sems[2:2 + NB]
    ss = sems[2 + NB]
    c = lax.axis_index("c")
    s = lax.axis_index("s")
    wid = c * NS + s
    row0 = s * RPT

    def _blk(b, slot):
        return (pltpu.make_async_copy(pk_hbm.at[wid, b], pk_v.at[slot], si[slot]),
                pltpu.make_async_copy(wt_hbm.at[wid, b], wt_v.at[slot], si[slot]))

    def _gather(bslot, jj, rslot):
        return pltpu.make_async_copy(
            x_hbm.at[pk_v.at[bslot, jj, 0]], rows_v.at[rslot], sg[rslot])

    def _scat_start(bslot, jj, rslot):
        pltpu.async_copy(
            rows_v.at[rslot], acc.at[pk_v.at[bslot, jj, 1]], ss, add=True)

    def _scat_wait(bslot, jj, rslot):
        pltpu.make_async_copy(
            rows_v.at[rslot], acc.at[pk_v.at[bslot, jj, 1]], ss).wait()

    for d in _blk(0, 0):
        d.start()

    def _zrow(i, carry):
        for g in range(VG):
            rows_v[0, i, pl.ds(g * 16, 16)] = jnp.zeros((16,), jnp.float32)
        return carry
    lax.fori_loop(0, CH, _zrow, 0)
    for k in range(ZC):
        pltpu.async_copy(rows_v.at[0], acc.at[pl.ds(row0 + k * CH, CH)], ss)
    for k in range(ZC):
        pltpu.make_async_copy(rows_v.at[0], acc.at[pl.ds(row0, CH)], ss).wait()
    plsc.subcore_barrier()

    for d in _blk(0, 0):
        d.wait()
    for jj in range(NB - 1):
        _gather(0, jj, jj).start()

    def _scale(bslot, jj, rslot):
        def body(e16, carry):
            base = e16 * 16
            wv = wt_v[bslot, jj, pl.ds(base, 16)]
            for i in range(16):
                w = wv[i]
                for g in range(VG):
                    sl = pl.ds(g * 16, 16)
                    rows_v[rslot, base + i, sl] = (
                        rows_v[rslot, base + i, sl] * w)
            return carry
        lax.fori_loop(0, CH // 16, body, 0)

    def _phase(b, sb, jj):
        j = b * BCH + jj
        rslot = jj % NB
        _gather(sb, jj, rslot).wait()
        _scale(sb, jj, rslot)

        jp = jj - 1 if jj >= 1 else BCH - 1
        sp = sb if jj >= 1 else 1 - sb
        rp = (jj - 1) % NB

        @pl.when(j >= 1)
        def _():
            _scat_wait(sp, jp, rp)

        if jj == 0:
            @pl.when(b + 1 < NBL)
            def _():
                for d in _blk(b + 1, 1 - sb):
                    d.start()

        if jj == BCH - NB + 1:
            @pl.when(b + 1 < NBL)
            def _():
                for d in _blk(b + 1, 1 - sb):
                    d.wait()

        jn = jj + NB - 1
        sn, jn = (sb, jn) if jn < BCH else (1 - sb, jn - BCH)

        @pl.when(j + NB - 1 < NCH)
        def _():
            _gather(sn, jn, (jj + NB - 1) % NB).start()

        _scat_start(sb, jj, rslot)

    def _group(g2, carry):
        for sb in range(2):
            b = g2 * 2 + sb
            for jj in range(BCH):
                _phase(b, sb, jj)
        return carry
    lax.fori_loop(0, NBL // 2, _group, 0)

    _scat_wait(1, BCH - 1, (NCH - 1) % NB)
    plsc.subcore_barrier()

    for k in range(ZC):
        pltpu.async_copy(
            acc.at[pl.ds(row0 + k * CH, CH)], out_hbm.at[c, pl.ds(row0 + k * CH, CH)], ss)
    for k in range(ZC):
        pltpu.make_async_copy(
            acc.at[pl.ds(row0, CH)], out_hbm.at[c, pl.ds(row0, CH)], ss).wait()


@functools.cache
def _build_spmm():
    return pl.kernel(
        _spmm_body,
        mesh=plsc.VectorSubcoreMesh(core_axis_name="c", subcore_axis_name="s"),
        out_type=jax.ShapeDtypeStruct((NC, NP, D), jnp.float32),
        scratch_types=[
            pltpu.VMEM((2, BCH, 2, CH), jnp.int32),
            pltpu.VMEM((2, BCH, CH), jnp.float32),
            pltpu.VMEM((NB, CH, D), jnp.float32),
            pltpu.VMEM_SHARED((NP, D), jnp.float32),
        ] + [pltpu.SemaphoreType.DMA] * (2 + NB + 1),
    )


def _transform(xb, w, b, sc, of):
    f = lax.dot_general(xb, w, (((1,), (1,)), ((), ())),
                        preferred_element_type=jnp.float32)
    f = jnp.maximum(f + b, 0.0)
    mean = jnp.mean(f, axis=1, keepdims=True)
    var = jnp.mean((f - mean) ** 2, axis=1, keepdims=True) + 1e-9
    return (f - mean) * sc * lax.rsqrt(var) + of


def _dense_a_body(p_ref, x_ref, w_ref, b_ref, s_ref, o_ref, h_ref, f_ref):
    h_ref[...] = p_ref[0] + p_ref[1]
    f_ref[...] = _transform(x_ref[...], w_ref[...], b_ref[...],
                            s_ref[...], o_ref[...])


def _dense_b_body(p_ref, h1_ref, f0_ref,
                  w1_ref, b1_ref, s1_ref, o1_ref,
                  w2_ref, b2_ref, s2_ref, o2_ref, out_ref):
    out_ref[:, 0:D] = f0_ref[...]
    out_ref[:, D:2 * D] = _transform(h1_ref[...], w1_ref[...], b1_ref[...],
                                     s1_ref[...], o1_ref[...])
    h2 = p_ref[0] + p_ref[1]
    out_ref[:, 2 * D:3 * D] = _transform(h2, w2_ref[...], b2_ref[...],
                                         s2_ref[...], o2_ref[...])


BR = 400
GR = N // BR


def _full(shape):
    return pl.BlockSpec(shape, lambda i: tuple(0 for _ in shape))


_rows2 = pl.BlockSpec((BR, D), lambda i: (i, 0))
_prows = pl.BlockSpec((2, BR, D), lambda i: (0, i, 0))

_dense_a = pl.pallas_call(
    _dense_a_body,
    grid=(GR,),
    in_specs=[_prows, _rows2, _full((D, D)),
              _full((1, D)), _full((1, D)), _full((1, D))],
    out_specs=[_rows2, _rows2],
    out_shape=[jax.ShapeDtypeStruct((N, D), jnp.float32),
               jax.ShapeDtypeStruct((N, D), jnp.float32)],
)

_dense_b = pl.pallas_call(
    _dense_b_body,
    grid=(GR,),
    in_specs=[_prows, _rows2, _rows2,
              _full((D, D)), _full((1, D)), _full((1, D)), _full((1, D)),
              _full((D, D)), _full((1, D)), _full((1, D)), _full((1, D))],
    out_specs=pl.BlockSpec((BR, 3 * D), lambda i: (i, 0)),
    out_shape=jax.ShapeDtypeStruct((N, 3 * D), jnp.float32),
)


def kernel(x, edge_index, edge_weight, W0, W1, W2,
           b0, b1, b2, s0, s1, s2, o0, o1, o2):
    pad = EP - E
    dst = jnp.concatenate(
        [edge_index[0].astype(jnp.int32), jnp.zeros((pad,), jnp.int32)])
    src = jnp.concatenate(
        [edge_index[1].astype(jnp.int32), jnp.zeros((pad,), jnp.int32)])
    wt = jnp.concatenate(
        [edge_weight, jnp.zeros((pad,), jnp.float32)]).reshape(NW, NBL, BCH, CH)
    pk = jnp.stack([src.reshape(NW, NCH, CH), dst.reshape(NW, NCH, CH)],
                   axis=2).reshape(NW, NBL, BCH, 2, CH)

    b0r, s0r, o0r = b0.reshape(1, D), s0.reshape(1, D), o0.reshape(1, D)
    b1r, s1r, o1r = b1.reshape(1, D), s1.reshape(1, D), o1.reshape(1, D)
    b2r, s2r, o2r = b2.reshape(1, D), s2.reshape(1, D), o2.reshape(1, D)

    spmm = _build_spmm()
    p1 = spmm(x, pk, wt)
    h1, f0 = _dense_a(p1, x, W0, b0r, s0r, o0r)
    p2 = spmm(h1, pk, wt)
    return _dense_b(p2, h1, f0, W1, b1r, s1r, o1r, W2, b2r, s2r, o2r)


# speedup vs baseline: 1.7439x; 1.7439x over previous
import functools

import jax
import jax.numpy as jnp
from jax import lax
from jax.experimental import pallas as pl
from jax.experimental.pallas import tpu as pltpu
from jax.experimental.pallas import tpu_sc as plsc

N = 10000
D = 128
E = 320000
NC = 2
NS = 16
NW = NC * NS
CH = 112
NCH = 90
EPW = NCH * CH
EP = NW * EPW
NP = 10240
RPT = NP // NS
VG = D // 16
NB = 3


def _spmm_body(x_hbm, pk_hbm, wt_hbm, out_hbm, pk_v, wt_v, rows_v, acc, *sems):
    si = sems[0:NB]
    sg = sems[NB:2 * NB]
    ss = sems[2 * NB]
    c = lax.axis_index("c")
    s = lax.axis_index("s")
    wid = c * NS + s
    row0 = s * RPT

    def _idx(j, slot, sem):
        return pltpu.make_async_copy(pk_hbm.at[wid, j], pk_v.at[slot], sem)

    def _wts(j, slot, sem):
        return pltpu.make_async_copy(wt_hbm.at[wid, j], wt_v.at[slot], sem)

    def _gather(slot, sem):
        return pltpu.make_async_copy(
            x_hbm.at[pk_v.at[slot, 0]], rows_v.at[slot], sem)


    _idx(0, 0, si[0]).start()
    _wts(0, 0, si[0]).start()
    _idx(1, 1, si[1]).start()
    _wts(1, 1, si[1]).start()

    def _zrow(i, carry):
        for g in range(VG):
            rows_v[0, i, pl.ds(g * 16, 16)] = jnp.zeros((16,), jnp.float32)
        return carry
    lax.fori_loop(0, CH, _zrow, 0)
    for k in range(RPT // CH):
        pltpu.sync_copy(rows_v.at[0], acc.at[pl.ds(row0 + k * CH, CH)])
    if RPT % CH:
        pltpu.sync_copy(
            rows_v.at[0, pl.ds(0, RPT % CH)],
            acc.at[pl.ds(row0 + (RPT // CH) * CH, RPT % CH)])
    plsc.subcore_barrier()

    _idx(0, 0, si[0]).wait()
    _wts(0, 0, si[0]).wait()
    _gather(0, sg[0]).start()

    def _scale(slot):
        def body(e16, carry):
            base = e16 * 16
            wv = wt_v[slot, pl.ds(base, 16)]
            for i in range(16):
                w = wv[i]
                for g in range(VG):
                    sl = pl.ds(g * 16, 16)
                    rows_v[slot, base + i, sl] = rows_v[slot, base + i, sl] * w
            return carry
        lax.fori_loop(0, CH // 16, body, 0)

    def _phase(j, cur, nxt, nn):
        @pl.when(j + 1 < NCH)
        def _():
            _idx(j + 1, nxt, si[nxt]).wait()
            _wts(j + 1, nxt, si[nxt]).wait()
            _gather(nxt, sg[nxt]).start()

        _gather(cur, sg[cur]).wait()
        _scale(cur)

        @pl.when(j >= 1)
        def _():
            pltpu.make_async_copy(
                rows_v.at[nn], acc.at[pk_v.at[nn, 1]], ss).wait()

        @pl.when(j + 2 < NCH)
        def _():
            _idx(j + 2, nn, si[nn]).start()
            _wts(j + 2, nn, si[nn]).start()

        pltpu.async_copy(
            rows_v.at[cur], acc.at[pk_v.at[cur, 1]], ss, add=True)

    def _group(g3, carry):
        j0 = g3 * NB
        for ph in range(NB):
            _phase(j0 + ph, ph, (ph + 1) % NB, (ph + 2) % NB)
        return carry
    lax.fori_loop(0, NCH // NB, _group, 0)

    last = (NCH - 1) % NB
    pltpu.make_async_copy(rows_v.at[last], acc.at[pk_v.at[last, 1]], ss).wait()
    plsc.subcore_barrier()

    def _ocopy(k, carry):
        sl = pl.ds(row0 + k * CH, CH)
        pltpu.sync_copy(acc.at[sl], out_hbm.at[c, sl])
        return carry
    lax.fori_loop(0, RPT // CH, _ocopy, 0)
    if RPT % CH:
        slr = pl.ds(row0 + (RPT // CH) * CH, RPT % CH)
        pltpu.sync_copy(acc.at[slr], out_hbm.at[c, slr])


@functools.cache
def _build_spmm():
    return pl.kernel(
        _spmm_body,
        mesh=plsc.VectorSubcoreMesh(core_axis_name="c", subcore_axis_name="s"),
        out_type=jax.ShapeDtypeStruct((NC, NP, D), jnp.float32),
        scratch_types=[
            pltpu.VMEM((NB, 2, CH), jnp.int32),
            pltpu.VMEM((NB, CH), jnp.float32),
            pltpu.VMEM((NB, CH, D), jnp.float32),
            pltpu.VMEM_SHARED((NP, D), jnp.float32),
        ] + [pltpu.SemaphoreType.DMA] * (2 * NB + 1),
    )


def _transform(xb, w, b, sc, of):
    f = lax.dot_general(xb, w, (((1,), (1,)), ((), ())),
                        preferred_element_type=jnp.float32)
    f = jnp.maximum(f + b, 0.0)
    mean = jnp.mean(f, axis=1, keepdims=True)
    var = jnp.mean((f - mean) ** 2, axis=1, keepdims=True) + 1e-9
    return (f - mean) * sc * lax.rsqrt(var) + of


def _dense_a_body(p_ref, x_ref, w_ref, b_ref, s_ref, o_ref, h_ref, f_ref):
    h_ref[...] = p_ref[0] + p_ref[1]
    f_ref[...] = _transform(x_ref[...], w_ref[...], b_ref[...],
                            s_ref[...], o_ref[...])


def _dense_b_body(p_ref, h1_ref, f0_ref,
                  w1_ref, b1_ref, s1_ref, o1_ref,
                  w2_ref, b2_ref, s2_ref, o2_ref, out_ref):
    out_ref[:, 0:D] = f0_ref[...]
    out_ref[:, D:2 * D] = _transform(h1_ref[...], w1_ref[...], b1_ref[...],
                                     s1_ref[...], o1_ref[...])
    h2 = p_ref[0] + p_ref[1]
    out_ref[:, 2 * D:3 * D] = _transform(h2, w2_ref[...], b2_ref[...],
                                         s2_ref[...], o2_ref[...])


BR = 400
GR = N // BR


def _full(shape):
    return pl.BlockSpec(shape, lambda i: tuple(0 for _ in shape))


_rows2 = pl.BlockSpec((BR, D), lambda i: (i, 0))
_prows = pl.BlockSpec((2, BR, D), lambda i: (0, i, 0))

_dense_a = pl.pallas_call(
    _dense_a_body,
    grid=(GR,),
    in_specs=[_prows, _rows2, _full((D, D)),
              _full((1, D)), _full((1, D)), _full((1, D))],
    out_specs=[_rows2, _rows2],
    out_shape=[jax.ShapeDtypeStruct((N, D), jnp.float32),
               jax.ShapeDtypeStruct((N, D), jnp.float32)],
)

_dense_b = pl.pallas_call(
    _dense_b_body,
    grid=(GR,),
    in_specs=[_prows, _rows2, _rows2,
              _full((D, D)), _full((1, D)), _full((1, D)), _full((1, D)),
              _full((D, D)), _full((1, D)), _full((1, D)), _full((1, D))],
    out_specs=pl.BlockSpec((BR, 3 * D), lambda i: (i, 0)),
    out_shape=jax.ShapeDtypeStruct((N, 3 * D), jnp.float32),
)


def kernel(x, edge_index, edge_weight, W0, W1, W2,
           b0, b1, b2, s0, s1, s2, o0, o1, o2):
    pad = EP - E
    dst = jnp.concatenate(
        [edge_index[0].astype(jnp.int32), jnp.zeros((pad,), jnp.int32)])
    src = jnp.concatenate(
        [edge_index[1].astype(jnp.int32), jnp.zeros((pad,), jnp.int32)])
    wt = jnp.concatenate(
        [edge_weight, jnp.zeros((pad,), jnp.float32)]).reshape(NW, NCH, CH)
    pk = jnp.stack([src.reshape(NW, NCH, CH), dst.reshape(NW, NCH, CH)],
                   axis=2)

    b0r, s0r, o0r = b0.reshape(1, D), s0.reshape(1, D), o0.reshape(1, D)
    b1r, s1r, o1r = b1.reshape(1, D), s1.reshape(1, D), o1.reshape(1, D)
    b2r, s2r, o2r = b2.reshape(1, D), s2.reshape(1, D), o2.reshape(1, D)

    spmm = _build_spmm()
    p1 = spmm(x, pk, wt)
    h1, f0 = _dense_a(p1, x, W0, b0r, s0r, o0r)
    p2 = spmm(h1, pk, wt)
    return _dense_b(p2, h1, f0, W1, b1r, s1r, o1r, W2, b2r, s2r, o2r)
